# baseline (speedup 1.0000x reference)
import functools

import jax
import jax.numpy as jnp
from jax import lax
from jax.experimental import pallas as pl
from jax.experimental.pallas import tpu as pltpu
from jax.experimental.pallas import tpu_sc as plsc

_N = 10000
_D = 128
_E = 320000
_EPS = 1e-5

_NC = 2
_NS = 16
_NT = _NC * _NS
_CH = 64
_CPT = 160
_EPT = _CPT * _CH
_EPAD = _NT * _EPT
_SLOTS = 4
_NROW = 10112
_RPT = _NROW // _NS
_NPAD = 10112
_DWC = _NPAD // 128


def _edge_weights(edge_attr, w0, b0, w1, b1):
    bE = 10000

    def kern(ea_ref, w0_ref, b0_ref, w1_ref, b1_ref, out_ref):
        ea = ea_ref[...]
        z0 = jnp.dot(ea, w0_ref[...], preferred_element_type=jnp.float32) + b0_ref[...]
        z1 = jnp.dot(ea, w1_ref[...], preferred_element_type=jnp.float32) + b1_ref[...]
        z = jnp.concatenate([z0, z1], axis=1)
        out_ref[...] = jnp.maximum(z, 0.0) + jnp.log1p(jnp.exp(-jnp.abs(z)))

    return pl.pallas_call(
        kern,
        grid=(_E // bE,),
        in_specs=[
            pl.BlockSpec((bE, 16), lambda i: (i, 0)),
            pl.BlockSpec((16, 1), lambda i: (0, 0)),
            pl.BlockSpec((1, 1), lambda i: (0, 0)),
            pl.BlockSpec((16, 1), lambda i: (0, 0)),
            pl.BlockSpec((1, 1), lambda i: (0, 0)),
        ],
        out_specs=pl.BlockSpec((bE, 2), lambda i: (i, 0)),
        out_shape=jax.ShapeDtypeStruct((_E, 2), jnp.float32),
    )(edge_attr, w0, b0.reshape(1, 1), w1, b1.reshape(1, 1))


def _sc_scatter(x, pck_t, wpk_t):
    mesh = plsc.VectorSubcoreMesh(core_axis_name="c", subcore_axis_name="s")

    @functools.partial(
        pl.kernel,
        mesh=mesh,
        out_type=(
            jax.ShapeDtypeStruct((_NC, _NROW, _D), jnp.float32),
            jax.ShapeDtypeStruct((_NC * _NPAD,), jnp.float32),
        ),
        scratch_types=[
            pltpu.VMEM((_CPT // 2, 128), jnp.int32),
            pltpu.VMEM((_CPT // 4, 128), jnp.int32),
            pltpu.VMEM((_SLOTS * _CH,), jnp.int32),
            pltpu.VMEM((_SLOTS, _CH), jnp.int32),
            pltpu.VMEM((_SLOTS * _CH,), jnp.float32),
            pltpu.VMEM((_SLOTS, _CH, _D), jnp.float32),
            pltpu.VMEM_SHARED((_NROW, _D), jnp.float32),
            pltpu.VMEM_SHARED((_NPAD,), jnp.float32),
            pltpu.SemaphoreType.DMA,
            pltpu.SemaphoreType.DMA,
            pltpu.SemaphoreType.DMA,
        ],
    )
    def k(x_hbm, pck_hbm, wpk_hbm, out_hbm, dw_hbm,
          pckb, wpkb, sidx, didx, wf, rowsb, acc_s, dw_s,
          gsem, ssem, dwsem):
        cid = lax.axis_index("c")
        sid = lax.axis_index("s")
        wid = cid * _NS + sid

        zero16 = jnp.zeros((16,), jnp.float32)

        def zrow(r, c):
            for j in range(_D // 16):
                rowsb[0, r, pl.ds(j * 16, 16)] = zero16
            return c
        lax.fori_loop(0, _CH, zrow, 0)

        nz = 0
        for t in range(_RPT // _CH):
            pltpu.async_copy(rowsb.at[0],
                             acc_s.at[pl.ds(sid * _RPT + t * _CH, _CH)],
                             ssem)
            nz += 1
        rem = _RPT % _CH
        if rem:
            pltpu.async_copy(
                rowsb.at[0, pl.ds(0, rem)],
                acc_s.at[pl.ds(sid * _RPT + (_RPT // _CH) * _CH, rem)],
                ssem)
        ndz = 0
        for t in range(-(-_DWC // _NS)):
            ci = t * _NS
            @pl.when(ci + sid < _DWC)
            def _():
                pltpu.async_copy(
                    rowsb.at[0, 0], dw_s.at[pl.ds((ci + sid) * 128, 128)],
                    dwsem)
            ndz += 1
        for t in range(nz):
            pltpu.make_async_copy(
                rowsb.at[0], acc_s.at[pl.ds(sid * _RPT, _CH)], ssem).wait()
        if rem:
            pltpu.make_async_copy(
                rowsb.at[0, pl.ds(0, rem)],
                acc_s.at[pl.ds(sid * _RPT, rem)], ssem).wait()
        for t in range(-(-_DWC // _NS)):
            ci = t * _NS
            @pl.when(ci + sid < _DWC)
            def _():
                pltpu.make_async_copy(
                    rowsb.at[0, 0], dw_s.at[pl.ds(0, 128)], dwsem).wait()
        plsc.subcore_barrier()

        pltpu.sync_copy(pck_hbm.at[wid], pckb)
        pltpu.sync_copy(wpk_hbm.at[wid], wpkb)

        def unpack_idx(row, colbase, slot):
            for u in range(_CH // 16):
                pv = pckb[row, pl.ds(colbase + u * 16, 16)]
                sidx[pl.ds(slot * _CH + u * 16, 16)] = pv & 16383
                didx[slot, pl.ds(u * 16, 16)] = pv >> 14

        def unpack_w(row, colbase, slot):
            for q in range(_CH // 32):
                wv = wpkb[row, pl.ds(colbase + q * 16, 16)]
                lo = lax.bitcast_convert_type(wv << 16, jnp.float32)
                hi = lax.bitcast_convert_type(wv & jnp.int32(-65536),
                                              jnp.float32)
                wf[pl.ds(slot * _CH + q * 32, 16)] = lo
                wf[pl.ds(slot * _CH + q * 32 + 16, 16)] = hi

        def gidx(slot):
            return sidx.at[pl.ds(slot * _CH, _CH)]

        unpack_idx(0, 0, 0)
        unpack_idx(0, _CH, 1)
        pltpu.async_copy(x_hbm.at[gidx(0)], rowsb.at[0], gsem)
        pltpu.async_copy(x_hbm.at[gidx(1)], rowsb.at[1], gsem)

        def quad(p, c):
            for b in range(4):
                i = p * 4 + b
                s0 = b
                s2 = (b + 2) % 4

                pltpu.make_async_copy(
                    x_hbm.at[gidx(s0)], rowsb.at[s0], gsem).wait()

                unpack_w(p, b * (_CH // 2), s0)

                def grp(g, c2):
                    wv = wf[pl.ds(s0 * _CH + g * 16, 16)]
                    for kk in range(16):
                        ws = wv[kk]
                        e = g * 16 + kk
                        for j in range(_D // 16):
                            sl = pl.ds(j * 16, 16)
                            rowsb[s0, e, sl] = rowsb[s0, e, sl] * ws
                    return c2
                lax.fori_loop(0, _CH // 16, grp, 0)

                pltpu.async_copy(rowsb.at[s0], acc_s.at[didx.at[s0]],
                                 ssem, add=True)
                pltpu.async_copy(wf.at[pl.ds(s0 * _CH, _CH)],
                                 dw_s.at[didx.at[s0]], dwsem, add=True)

                @pl.when(i >= 2)
                def _():
                    pltpu.make_async_copy(
                        rowsb.at[s2], acc_s.at[didx.at[s2]], ssem).wait()
                    pltpu.make_async_copy(
                        wf.at[pl.ds(s2 * _CH, _CH)], dw_s.at[didx.at[s2]],
                        dwsem).wait()

                @pl.when(i + 2 < _CPT)
                def _():
                    unpack_idx(2 * p + (b + 2) // 2, ((b + 2) % 2) * _CH, s2)
                    pltpu.async_copy(
                        x_hbm.at[gidx(s2)], rowsb.at[s2], gsem)
            return c
        lax.fori_loop(0, _CPT // 4, quad, 0)

        for i in (_CPT - 2, _CPT - 1):
            s = i % 4
            pltpu.make_async_copy(
                rowsb.at[s], acc_s.at[didx.at[s]], ssem).wait()
            pltpu.make_async_copy(
                wf.at[pl.ds(s * _CH, _CH)], dw_s.at[didx.at[s]],
                dwsem).wait()

        plsc.subcore_barrier()
        pltpu.sync_copy(acc_s.at[pl.ds(sid * _RPT, _RPT)],
                        out_hbm.at[cid, pl.ds(sid * _RPT, _RPT)])
        for t in range(-(-_DWC // _NS)):
            ci = t * _NS
            @pl.when(ci + sid < _DWC)
            def _():
                pltpu.sync_copy(
                    dw_s.at[pl.ds((ci + sid) * 128, 128)],
                    dw_hbm.at[pl.ds(cid * _NPAD + (ci + sid) * 128, 128)])

    return k(x, pck_t, wpk_t)


def _combine(p0, p1, dw0, dw1, xin, lin_W, lin_b, ln_g, ln_bt,
             fc_W=None, fc_b=None):
    bN = 1000
    final = fc_W is not None

    def kern(*refs):
        if final:
            (p0_ref, p1_ref, dw0_ref, dw1_ref, x_ref, w_ref, b_ref,
             g_ref, bt_ref, fw_ref, fb_ref, out_ref) = refs
        else:
            (p0_ref, p1_ref, dw0_ref, dw1_ref, x_ref, w_ref, b_ref,
             g_ref, bt_ref, out_ref) = refs
        dw = dw0_ref[...] + dw1_ref[...]
        aggr = p0_ref[...] + p1_ref[...] - dw * x_ref[...]
        h = lax.dot_general(aggr, w_ref[...], (((1,), (1,)), ((), ())),
                            preferred_element_type=jnp.float32) + b_ref[...]
        h = jnp.maximum(h, 0.0)
        mu = jnp.mean(h, axis=1, keepdims=True)
        hc = h - mu
        var = jnp.mean(hc * hc, axis=1, keepdims=True)
        hn = hc * lax.rsqrt(var + _EPS) * g_ref[...] + bt_ref[...]
        if final:
            hn = lax.dot_general(hn, fw_ref[...], (((1,), (1,)), ((), ())),
                                 preferred_element_type=jnp.float32) + fb_ref[...]
        out_ref[...] = hn

    row = pl.BlockSpec((bN, _D), lambda i: (i, 0))
    col = pl.BlockSpec((bN, 1), lambda i: (i, 0))
    full = pl.BlockSpec((_D, _D), lambda i: (0, 0))
    vec = pl.BlockSpec((1, _D), lambda i: (0, 0))
    in_specs = [row, row, col, col, row, full, vec, vec, vec]
    args = [p0, p1, dw0, dw1, xin, lin_W, lin_b.reshape(1, _D),
            ln_g.reshape(1, _D), ln_bt.reshape(1, _D)]
    if final:
        in_specs += [full, vec]
        args += [fc_W, fc_b.reshape(1, _D)]

    return pl.pallas_call(
        kern,
        grid=(_N // bN,),
        in_specs=in_specs,
        out_specs=row,
        out_shape=jax.ShapeDtypeStruct((_N, _D), jnp.float32),
    )(*args)


def kernel(x, edge_index, edge_attr, lin0_W, lin0_b, emlp0_W, emlp0_b,
           ln0_g, ln0_bt, lin1_W, lin1_b, emlp1_W, emlp1_b, ln1_g, ln1_bt,
           fc_W, fc_b):
    src = edge_index[0]
    dst = edge_index[1]

    w01 = _edge_weights(edge_attr, emlp0_W, emlp0_b, emlp1_W, emlp1_b)

    pad = _EPAD - _E
    pck = dst * 16384 + src
    pck_t = jnp.pad(pck, (0, pad)).reshape(_NT, _CPT // 2, 128)

    def pack_w(w):
        wt = jnp.pad(w, (0, pad)).reshape(_NT, _EPT // 32, 2, 16)
        bits = lax.bitcast_convert_type(
            wt.astype(jnp.bfloat16), jnp.uint16).astype(jnp.uint32)
        words = bits[:, :, 0, :] | (bits[:, :, 1, :] << 16)
        return lax.bitcast_convert_type(words, jnp.int32).reshape(
            _NT, _CPT // 4, 128)

    w0_t = pack_w(w01[:, 0])
    w1_t = pack_w(w01[:, 1])

    p, dwp = _sc_scatter(x, pck_t, w0_t)
    dwp = dwp.reshape(_NC, _NPAD)
    dw0 = dwp[0, :_N].reshape(_N, 1)
    dw1 = dwp[1, :_N].reshape(_N, 1)
    h = _combine(p[0, :_N], p[1, :_N], dw0, dw1, x,
                 lin0_W, lin0_b, ln0_g, ln0_bt)

    p, dwp = _sc_scatter(h, pck_t, w1_t)
    dwp = dwp.reshape(_NC, _NPAD)
    dw0 = dwp[0, :_N].reshape(_N, 1)
    dw1 = dwp[1, :_N].reshape(_N, 1)
    return _combine(p[0, :_N], p[1, :_N], dw0, dw1, h,
                    lin1_W, lin1_b, ln1_g, ln1_bt, fc_W, fc_b)
